# baseline (speedup 1.0000x reference)
import functools

import jax
import jax.numpy as jnp
from jax import lax
from jax.experimental import pallas as pl
from jax.experimental.pallas import tpu as pltpu
from jax.experimental.pallas import tpu_sc as plsc

D_MODEL = 1024
SEQ = 8192
BATCH = 2

_SPINE = [0, 2, 4, 12, 36, 104, 304, 888, 2592, 7568]
_WRITE_POS = _SPINE[3:]
_NW = len(_WRITE_POS)
_NSP = len(_SPINE)

_CH = 2048
_NS = 4
_LAG = 2

_CHUNKS = []
_SINGLES = []
for _b in range(BATCH):
    _prev = _b * SEQ
    for _p in [_b * SEQ + q for q in _WRITE_POS] + [(_b + 1) * SEQ]:
        _s, _e = _prev, _p
        if _s < _e:
            _sa = min(_e, (_s + 7) // 8 * 8)
            _ea = max(_sa, _e // 8 * 8)
            _SINGLES.extend(range(_s, _sa))
            _m = _sa
            while _m < _ea:
                _l = min(_CH, _ea - _m)
                _CHUNKS.append((_m, _l))
                _m += _l
            _SINGLES.extend(range(_ea, _e))
        _prev = _p + 1
_NSEG = len(_CHUNKS)

_SC_JOBS = [(0, r, r) for r in _SINGLES]
_SC_JOBS += [(1, b * _NW + j, b * SEQ + p)
             for b in range(BATCH) for j, p in enumerate(_WRITE_POS)]
_NWORKERS = 32


def _recur_kernel(x_ref, w_ref, axz_ref, bxy_ref, gx_ref, awy_ref,
                  bwx_ref, gw_ref, axv_ref, bwv_ref, gv_ref,
                  gb_ref, lnw_ref, lnb_ref, out_ref,
                  rows_v, wmat, gat_sem, w_sem):
    def row_gather(b, i):
        return pltpu.make_async_copy(
            x_ref.at[pl.ds(b * SEQ + _SPINE[i], 1), :],
            rows_v.at[pl.ds(b * _NSP + i, 1), :],
            gat_sem)

    w_copy = pltpu.make_async_copy(w_ref, wmat, w_sem)
    for b in range(BATCH):
        for i in range(_NSP):
            row_gather(b, i).start()
    w_copy.start()

    av = axv_ref[...] * axz_ref[...]
    bv = axv_ref[...] * bxy_ref[...] + bwv_ref[...] * awy_ref[...]
    cv = bwv_ref[...] * bwx_ref[...]
    dv = axv_ref[...] * gx_ref[...] + bwv_ref[...] * gw_ref[...] + gv_ref[...]
    gb = gb_ref[...]
    lnw = lnw_ref[...]
    lnb = lnb_ref[...]

    for b in range(BATCH):
        for i in range(_NSP):
            row_gather(b, i).wait()
    w_copy.wait()
    w_vb = wmat[:, :D_MODEL].astype(jnp.bfloat16)
    w_zb = wmat[:, D_MODEL:].astype(jnp.bfloat16)
    dims = (((1,), (1,)), ((), ()))

    def vrow(k):
        return jnp.concatenate(
            [rows_v[k:k + 1, :], rows_v[_NSP + k:_NSP + k + 1, :]], axis=0)

    vals = [vrow(k) for k in range(_NSP)]

    z_all = jnp.concatenate(
        [rows_v[3:_NSP, :], rows_v[_NSP + 3:, :]], axis=0)
    aug = jnp.concatenate([av * z_all, dv[None, :]], axis=0)
    p1 = jax.lax.dot_general(aug.astype(jnp.bfloat16), w_vb, dims,
                             preferred_element_type=jnp.float32)
    p2 = jax.lax.dot_general(z_all.astype(jnp.bfloat16), w_zb, dims,
                             preferred_element_type=jnp.float32)
    base = p1[:BATCH * _NW] + p2 + p1[BATCH * _NW:] + gb

    for k in range(3, _NSP):
        j = k - 3
        z = vals[k]
        y = vals[k - 1]
        x_prev = vals[k - 2]
        t = bv * y + cv * x_prev
        q = jax.lax.dot_general(t.astype(jnp.bfloat16), w_vb, dims,
                                preferred_element_type=jnp.float32)
        logits = q + jnp.concatenate(
            [base[j:j + 1], base[_NW + j:_NW + j + 1]], axis=0)
        g = jax.nn.sigmoid(logits)
        v = av * z + t + dv
        gated = g * v + (1.0 - g) * z
        mean = jnp.mean(gated, axis=-1, keepdims=True)
        var = jnp.mean((gated - mean) ** 2, axis=-1, keepdims=True)
        vals[k] = (gated - mean) * jax.lax.rsqrt(var + 1e-5) * lnw + lnb
        for b in range(BATCH):
            out_ref[b * _NW + j, :] = vals[k][b, :]


def _sc_rows_body(x_ref, rows_ref, out_ref, buf):
    cid = lax.axis_index("c")
    sid = lax.axis_index("s")
    wid = sid * 2 + cid
    for w in range(_NWORKERS):
        jobs = _SC_JOBS[w::_NWORKERS]
        if not jobs:
            continue

        @pl.when(wid == w)
        def _(jobs=jobs):
            for src, sr, dr in jobs:
                ref = x_ref if src == 0 else rows_ref
                pltpu.sync_copy(ref.at[pl.ds(sr, 1), :], buf)
                pltpu.sync_copy(buf, out_ref.at[pl.ds(dr, 1), :])


def _copy_kernel(x_ref, pre_ref, out_ref, buf, in_sems, out_sems):
    del pre_ref

    def in_copy(i):
        s, l = _CHUNKS[i]
        return pltpu.make_async_copy(
            x_ref.at[pl.ds(s, l), :], buf.at[i % _NS, pl.ds(0, l), :],
            in_sems.at[i % _NS])

    def out_copy(i):
        s, l = _CHUNKS[i]
        return pltpu.make_async_copy(
            buf.at[i % _NS, pl.ds(0, l), :], out_ref.at[pl.ds(s, l), :],
            out_sems.at[i % _NS])

    for i in range(min(_NS, _NSEG)):
        in_copy(i).start()
    waited = set()
    for i in range(_NSEG):
        in_copy(i).wait()
        out_copy(i).start()
        w = i - _LAG
        n = w + _NS
        if w >= 0 and n < _NSEG:
            out_copy(w).wait()
            waited.add(w)
            in_copy(n).start()
    for i in range(_NSEG):
        if i not in waited:
            out_copy(i).wait()


def kernel(x, alpha_xz, beta_xy, gamma_x, alpha_wy, beta_wx, gamma_w,
           alpha_xv, beta_wv, gamma_v, gate_w, gate_b, ln_w, ln_b):
    x_flat = x.reshape(BATCH * SEQ, D_MODEL)

    hspec = pl.BlockSpec(memory_space=pltpu.MemorySpace.HBM)
    vspec = pl.BlockSpec(memory_space=pltpu.MemorySpace.VMEM)

    new_rows = pl.pallas_call(
        _recur_kernel,
        in_specs=[hspec, hspec] + [vspec] * 12,
        out_specs=vspec,
        out_shape=jax.ShapeDtypeStruct((BATCH * _NW, D_MODEL), jnp.float32),
        scratch_shapes=[
            pltpu.VMEM((BATCH * _NSP, D_MODEL), jnp.float32),
            pltpu.VMEM((D_MODEL, 2 * D_MODEL), jnp.float32),
            pltpu.SemaphoreType.DMA,
            pltpu.SemaphoreType.DMA,
        ],
    )(x_flat, gate_w, alpha_xz, beta_xy, gamma_x, alpha_wy, beta_wx,
      gamma_w, alpha_xv, beta_wv, gamma_v, gate_b, ln_w, ln_b)

    sc_rows = functools.partial(
        pl.kernel,
        out_type=jax.ShapeDtypeStruct((BATCH * SEQ, D_MODEL), jnp.float32),
        mesh=plsc.VectorSubcoreMesh(core_axis_name="c", subcore_axis_name="s"),
        scratch_types=[pltpu.VMEM((1, D_MODEL), jnp.float32)],
        compiler_params=pltpu.CompilerParams(use_tc_tiling_on_sc=True),
    )(_sc_rows_body)
    pre = sc_rows(x_flat, new_rows)

    out_flat = pl.pallas_call(
        _copy_kernel,
        in_specs=[hspec, hspec],
        out_specs=hspec,
        out_shape=jax.ShapeDtypeStruct((BATCH * SEQ, D_MODEL), jnp.float32),
        input_output_aliases={1: 0},
        scratch_shapes=[
            pltpu.VMEM((_NS, _CH, D_MODEL), jnp.float32),
            pltpu.SemaphoreType.DMA((_NS,)),
            pltpu.SemaphoreType.DMA((_NS,)),
        ],
    )(x_flat, pre)
    return out_flat.reshape(BATCH, SEQ, D_MODEL)


# speedup vs baseline: 1.5937x; 1.5937x over previous
import jax
import jax.numpy as jnp
from jax.experimental import pallas as pl
from jax.experimental.pallas import tpu as pltpu

D_MODEL = 1024
SEQ = 8192
BATCH = 2

_SPINE = [0, 2, 4, 12, 36, 104, 304, 888, 2592, 7568]
_WRITE_POS = _SPINE[3:]
_NW = len(_WRITE_POS)
_NSP = len(_SPINE)

_CH = 2048
_NC = (BATCH * SEQ) // _CH
_NS = 4
_LAG = 2


def _fused_kernel(x_ref, w_ref, axz_ref, bxy_ref, gx_ref, awy_ref,
                  bwx_ref, gw_ref, axv_ref, bwv_ref, gv_ref,
                  gb_ref, lnw_ref, lnb_ref, out_ref,
                  buf, rows_v, wmat, new_rows,
                  in_sems, out_sems, gat_sem, w_sem, scat_sem):
    def in_copy(c):
        return pltpu.make_async_copy(
            x_ref.at[pl.ds(c * _CH, _CH), :], buf.at[c % _NS],
            in_sems.at[c % _NS])

    def out_copy(c):
        return pltpu.make_async_copy(
            buf.at[c % _NS], out_ref.at[pl.ds(c * _CH, _CH), :],
            out_sems.at[c % _NS])

    def row_gather(b, i):
        return pltpu.make_async_copy(
            x_ref.at[pl.ds(b * SEQ + _SPINE[i], 1), :],
            rows_v.at[pl.ds(b * _NSP + i, 1), :],
            gat_sem)

    w_copy = pltpu.make_async_copy(w_ref, wmat, w_sem)

    for c in range(min(_NS, _NC)):
        in_copy(c).start()
    for b in range(BATCH):
        for i in range(_NSP):
            row_gather(b, i).start()
    w_copy.start()

    av = axv_ref[...] * axz_ref[...]
    bv = axv_ref[...] * bxy_ref[...] + bwv_ref[...] * awy_ref[...]
    cv = bwv_ref[...] * bwx_ref[...]
    dv = axv_ref[...] * gx_ref[...] + bwv_ref[...] * gw_ref[...] + gv_ref[...]
    gb = gb_ref[...]
    lnw = lnw_ref[...]
    lnb = lnb_ref[...]

    for b in range(BATCH):
        for i in range(_NSP):
            row_gather(b, i).wait()
    w_copy.wait()
    w_vb = wmat[:, :D_MODEL].astype(jnp.bfloat16)
    w_zb = wmat[:, D_MODEL:].astype(jnp.bfloat16)
    dims = (((1,), (1,)), ((), ()))

    def vrow(k):
        return jnp.concatenate(
            [rows_v[k:k + 1, :], rows_v[_NSP + k:_NSP + k + 1, :]], axis=0)

    vals = [vrow(k) for k in range(_NSP)]

    z_all = jnp.concatenate(
        [rows_v[3:_NSP, :], rows_v[_NSP + 3:, :]], axis=0)
    aug = jnp.concatenate([av * z_all, dv[None, :]], axis=0)
    p1 = jax.lax.dot_general(aug.astype(jnp.bfloat16), w_vb, dims,
                             preferred_element_type=jnp.float32)
    p2 = jax.lax.dot_general(z_all.astype(jnp.bfloat16), w_zb, dims,
                             preferred_element_type=jnp.float32)
    base = p1[:BATCH * _NW] + p2 + p1[BATCH * _NW:] + gb

    def recurrence_step(k):
        j = k - 3
        z = vals[k]
        y = vals[k - 1]
        x_prev = vals[k - 2]
        t = bv * y + cv * x_prev
        q = jax.lax.dot_general(t.astype(jnp.bfloat16), w_vb, dims,
                                preferred_element_type=jnp.float32)
        logits = q + jnp.concatenate(
            [base[j:j + 1], base[_NW + j:_NW + j + 1]], axis=0)
        g = jax.nn.sigmoid(logits)
        v = av * z + t + dv
        gated = g * v + (1.0 - g) * z
        mean = jnp.mean(gated, axis=-1, keepdims=True)
        var = jnp.mean((gated - mean) ** 2, axis=-1, keepdims=True)
        vals[k] = (gated - mean) * jax.lax.rsqrt(var + 1e-5) * lnw + lnb
        for b in range(BATCH):
            new_rows[b * _NW + j, :] = vals[k][b, :]

    waited = set()
    for c in range(_NC):
        in_copy(c).wait()
        out_copy(c).start()
        if 3 + c < _NSP:
            recurrence_step(3 + c)
        w = c - _LAG
        n = w + _NS
        if w >= 0 and n < _NC:
            out_copy(w).wait()
            waited.add(w)
            in_copy(n).start()
    for c in range(_NC):
        if c not in waited:
            out_copy(c).wait()

    for b in range(BATCH):
        for j, p in enumerate(_WRITE_POS):
            pltpu.make_async_copy(
                new_rows.at[pl.ds(b * _NW + j, 1), :],
                out_ref.at[pl.ds(b * SEQ + p, 1), :],
                scat_sem,
            ).start()
    for b in range(BATCH):
        for j, p in enumerate(_WRITE_POS):
            pltpu.make_async_copy(
                new_rows.at[pl.ds(b * _NW + j, 1), :],
                out_ref.at[pl.ds(b * SEQ + p, 1), :],
                scat_sem,
            ).wait()


def kernel(x, alpha_xz, beta_xy, gamma_x, alpha_wy, beta_wx, gamma_w,
           alpha_xv, beta_wv, gamma_v, gate_w, gate_b, ln_w, ln_b):
    x_flat = x.reshape(BATCH * SEQ, D_MODEL)

    hspec = pl.BlockSpec(memory_space=pltpu.MemorySpace.HBM)
    vspec = pl.BlockSpec(memory_space=pltpu.MemorySpace.VMEM)
    out_flat = pl.pallas_call(
        _fused_kernel,
        in_specs=[hspec, hspec] + [vspec] * 12,
        out_specs=hspec,
        out_shape=jax.ShapeDtypeStruct((BATCH * SEQ, D_MODEL), jnp.float32),
        scratch_shapes=[
            pltpu.VMEM((_NS, _CH, D_MODEL), jnp.float32),
            pltpu.VMEM((BATCH * _NSP, D_MODEL), jnp.float32),
            pltpu.VMEM((D_MODEL, 2 * D_MODEL), jnp.float32),
            pltpu.VMEM((BATCH * _NW, D_MODEL), jnp.float32),
            pltpu.SemaphoreType.DMA((_NS,)),
            pltpu.SemaphoreType.DMA((_NS,)),
            pltpu.SemaphoreType.DMA,
            pltpu.SemaphoreType.DMA,
            pltpu.SemaphoreType.DMA,
        ],
    )(x_flat, gate_w, alpha_xz, beta_xy, gamma_x, alpha_wy, beta_wx,
      gamma_w, alpha_xv, beta_wv, gamma_v, gate_b, ln_w, ln_b)
    return out_flat.reshape(BATCH, SEQ, D_MODEL)


# speedup vs baseline: 1.6072x; 1.0085x over previous
import jax
import jax.numpy as jnp
from jax.experimental import pallas as pl
from jax.experimental.pallas import tpu as pltpu

D_MODEL = 1024
SEQ = 8192
BATCH = 2

_SPINE = [0, 2, 4, 12, 36, 104, 304, 888, 2592, 7568]
_WRITE_POS = _SPINE[3:]
_NW = len(_WRITE_POS)
_NSP = len(_SPINE)

_CH = 2048
_NC = (BATCH * SEQ) // _CH
_NS = 5
_LAG = 2


def _fused_kernel(x_ref, w_ref, axz_ref, bxy_ref, gx_ref, awy_ref,
                  bwx_ref, gw_ref, axv_ref, bwv_ref, gv_ref,
                  gb_ref, lnw_ref, lnb_ref, out_ref,
                  buf, rows_v, wmat, new_rows,
                  in_sems, out_sems, gat_sem, w_sem, scat_sem):
    def in_copy(c):
        return pltpu.make_async_copy(
            x_ref.at[pl.ds(c * _CH, _CH), :], buf.at[c % _NS],
            in_sems.at[c % _NS])

    def out_copy(c):
        return pltpu.make_async_copy(
            buf.at[c % _NS], out_ref.at[pl.ds(c * _CH, _CH), :],
            out_sems.at[c % _NS])

    def row_gather(b, i):
        return pltpu.make_async_copy(
            x_ref.at[pl.ds(b * SEQ + _SPINE[i], 1), :],
            rows_v.at[pl.ds(b * _NSP + i, 1), :],
            gat_sem)

    w_copy = pltpu.make_async_copy(w_ref, wmat, w_sem)

    for c in range(min(_NS, _NC)):
        in_copy(c).start()
    for b in range(BATCH):
        for i in range(_NSP):
            row_gather(b, i).start()
    w_copy.start()

    av = axv_ref[...] * axz_ref[...]
    bv = axv_ref[...] * bxy_ref[...] + bwv_ref[...] * awy_ref[...]
    cv = bwv_ref[...] * bwx_ref[...]
    dv = axv_ref[...] * gx_ref[...] + bwv_ref[...] * gw_ref[...] + gv_ref[...]
    gb = gb_ref[...]
    lnw = lnw_ref[...]
    lnb = lnb_ref[...]

    for b in range(BATCH):
        for i in range(_NSP):
            row_gather(b, i).wait()
    w_copy.wait()
    w_vb = wmat[:, :D_MODEL].astype(jnp.bfloat16)
    w_zb = wmat[:, D_MODEL:].astype(jnp.bfloat16)
    dims = (((1,), (1,)), ((), ()))

    def vrow(k):
        return jnp.concatenate(
            [rows_v[k:k + 1, :], rows_v[_NSP + k:_NSP + k + 1, :]], axis=0)

    vals = [vrow(k) for k in range(_NSP)]

    z_all = jnp.concatenate(
        [rows_v[3:_NSP, :], rows_v[_NSP + 3:, :]], axis=0)
    aug = jnp.concatenate([av * z_all, dv[None, :]], axis=0)
    p1 = jax.lax.dot_general(aug.astype(jnp.bfloat16), w_vb, dims,
                             preferred_element_type=jnp.float32)
    p2 = jax.lax.dot_general(z_all.astype(jnp.bfloat16), w_zb, dims,
                             preferred_element_type=jnp.float32)
    base = p1[:BATCH * _NW] + p2 + p1[BATCH * _NW:] + gb

    def recurrence_step(k):
        j = k - 3
        z = vals[k]
        y = vals[k - 1]
        x_prev = vals[k - 2]
        t = bv * y + cv * x_prev
        q = jax.lax.dot_general(t.astype(jnp.bfloat16), w_vb, dims,
                                preferred_element_type=jnp.float32)
        logits = q + jnp.concatenate(
            [base[j:j + 1], base[_NW + j:_NW + j + 1]], axis=0)
        g = jax.nn.sigmoid(logits)
        v = av * z + t + dv
        gated = g * v + (1.0 - g) * z
        mean = jnp.mean(gated, axis=-1, keepdims=True)
        var = jnp.mean((gated - mean) ** 2, axis=-1, keepdims=True)
        vals[k] = (gated - mean) * jax.lax.rsqrt(var + 1e-5) * lnw + lnb
        for b in range(BATCH):
            new_rows[b * _NW + j, :] = vals[k][b, :]

    waited = set()
    for c in range(_NC):
        in_copy(c).wait()
        out_copy(c).start()
        if 3 + c < _NSP:
            recurrence_step(3 + c)
        w = c - _LAG
        n = w + _NS
        if w >= 0 and n < _NC:
            out_copy(w).wait()
            waited.add(w)
            in_copy(n).start()
    for c in range(_NC):
        if c not in waited:
            out_copy(c).wait()

    for b in range(BATCH):
        for j, p in enumerate(_WRITE_POS):
            pltpu.make_async_copy(
                new_rows.at[pl.ds(b * _NW + j, 1), :],
                out_ref.at[pl.ds(b * SEQ + p, 1), :],
                scat_sem,
            ).start()
    for b in range(BATCH):
        for j, p in enumerate(_WRITE_POS):
            pltpu.make_async_copy(
                new_rows.at[pl.ds(b * _NW + j, 1), :],
                out_ref.at[pl.ds(b * SEQ + p, 1), :],
                scat_sem,
            ).wait()


def kernel(x, alpha_xz, beta_xy, gamma_x, alpha_wy, beta_wx, gamma_w,
           alpha_xv, beta_wv, gamma_v, gate_w, gate_b, ln_w, ln_b):
    x_flat = x.reshape(BATCH * SEQ, D_MODEL)

    hspec = pl.BlockSpec(memory_space=pltpu.MemorySpace.HBM)
    vspec = pl.BlockSpec(memory_space=pltpu.MemorySpace.VMEM)
    out_flat = pl.pallas_call(
        _fused_kernel,
        in_specs=[hspec, hspec] + [vspec] * 12,
        out_specs=hspec,
        out_shape=jax.ShapeDtypeStruct((BATCH * SEQ, D_MODEL), jnp.float32),
        scratch_shapes=[
            pltpu.VMEM((_NS, _CH, D_MODEL), jnp.float32),
            pltpu.VMEM((BATCH * _NSP, D_MODEL), jnp.float32),
            pltpu.VMEM((D_MODEL, 2 * D_MODEL), jnp.float32),
            pltpu.VMEM((BATCH * _NW, D_MODEL), jnp.float32),
            pltpu.SemaphoreType.DMA((_NS,)),
            pltpu.SemaphoreType.DMA((_NS,)),
            pltpu.SemaphoreType.DMA,
            pltpu.SemaphoreType.DMA,
            pltpu.SemaphoreType.DMA,
        ],
    )(x_flat, gate_w, alpha_xz, beta_xy, gamma_x, alpha_wy, beta_wx,
      gamma_w, alpha_xv, beta_wv, gamma_v, gate_b, ln_w, ln_b)
    return out_flat.reshape(BATCH, SEQ, D_MODEL)
